# baseline (speedup 1.0000x reference)
import functools

import jax
import jax.numpy as jnp
from jax import lax
from jax.experimental import pallas as pl
from jax.experimental.pallas import tpu as pltpu
from jax.experimental.pallas import tpu_sc as plsc

_BS = 1024
_NCLS = 20
_NQ = 27
_CPAD = 32
_FLAT = _NCLS * _NQ
_L = 16

_NUM_CORES = 2
_NUM_SUBCORES = 16
_NW = _NUM_CORES * _NUM_SUBCORES
_BPW = _BS // _NW

_TGT_W = 6 * _CPAD
_ANCH_W = _NQ * _CPAD


def _sc_body(probs_hbm, tgt_hbm, anch_hbm, ntq_hbm,
             soft_out, match_out,
             probs_v, tgt_v, anch_v, b1lo_v, b1hi_v, v1_v, costg_v,
             soft_v, match_v, ntq_v):
    wid = lax.axis_index("s") * _NUM_CORES + lax.axis_index("c")
    base = wid * _BPW

    pltpu.sync_copy(probs_hbm.at[pl.ds(base * _FLAT, _BPW * _FLAT)], probs_v)
    pltpu.sync_copy(tgt_hbm.at[pl.ds(base * _TGT_W, _BPW * _TGT_W)], tgt_v)
    pltpu.sync_copy(anch_hbm, anch_v)
    pltpu.sync_copy(ntq_hbm, ntq_v)

    lane = lax.broadcasted_iota(jnp.int32, (_L,), 0)
    ntq = ntq_v[:]
    half = jnp.float32(0.5)

    def _pro(q, c0):
        for h in range(2):
            off = q * _CPAD + h * _L
            a = [anch_v[pl.ds(d * _ANCH_W + off, _L)] for d in range(6)]
            ac = [jnp.maximum(x, 0.0) for x in a]
            es = []
            for d in range(3):
                lo = ac[d] - half * ac[d + 3]
                hi = ac[d] + half * ac[d + 3]
                b1lo_v[pl.ds(d * _ANCH_W + off, _L)] = lo
                b1hi_v[pl.ds(d * _ANCH_W + off, _L)] = hi
                es.append(jnp.maximum(hi - lo, 0.0))
            v1_v[pl.ds(off, _L)] = es[0] * es[1] * es[2]
        return c0

    lax.fori_loop(0, _NQ, _pro, 0)

    def _batch(b, c0):
        for h in range(2):
            hoff = h * _L
            mask = None if h == 0 else (lane < (_NCLS - _L))
            cbase = (lane + hoff) * _NQ + b * _FLAT

            t = [tgt_v[pl.ds(b * _TGT_W + d * _CPAD + hoff, _L)]
                 for d in range(6)]
            b2lo = [t[d] - half * t[d + 3] for d in range(3)]
            b2hi = [t[d] + half * t[d + 3] for d in range(3)]
            e0 = jnp.maximum(b2hi[0] - b2lo[0], 0.0)
            e1 = jnp.maximum(b2hi[1] - b2lo[1], 0.0)
            e2 = jnp.maximum(b2hi[2] - b2lo[2], 0.0)
            v2 = e0 * e1 * e2

            inf = jnp.full((_L,), jnp.inf, dtype=jnp.float32)
            ninf = jnp.full((_L,), -jnp.inf, dtype=jnp.float32)
            zero_i = jnp.zeros((_L,), dtype=jnp.int32)

            def _q(q, carry):
                minval, minidx, gmin, gmax = carry
                iv = cbase + q
                if mask is None:
                    p = plsc.load_gather(probs_v, [iv])
                else:
                    p = plsc.load_gather(probs_v, [iv], mask=mask)
                qoff = q * _CPAD + hoff
                a = [anch_v[pl.ds(d * _ANCH_W + qoff, _L)] for d in range(6)]
                diff = [a[d] - t[d] for d in range(6)]
                bbox = (jnp.abs(diff[0]) + jnp.abs(diff[1]) + jnp.abs(diff[2])
                        + jnp.abs(diff[3]) + jnp.abs(diff[4]) + jnp.abs(diff[5]))
                ssq = diff[0] * diff[0] + diff[1] * diff[1] + diff[2] * diff[2]
                ssq = jnp.maximum(ssq, 1e-12)
                bits = lax.bitcast_convert_type(ssq, jnp.int32)
                y = lax.bitcast_convert_type(
                    (bits >> 1) + jnp.int32(0x1FBD1DF5), jnp.float32)
                y = half * (y + ssq / y)
                y = half * (y + ssq / y)
                center = half * (y + ssq / y)

                lo1 = [b1lo_v[pl.ds(d * _ANCH_W + qoff, _L)] for d in range(3)]
                hi1 = [b1hi_v[pl.ds(d * _ANCH_W + qoff, _L)] for d in range(3)]
                v1 = v1_v[pl.ds(qoff, _L)]
                iw0 = jnp.maximum(jnp.minimum(hi1[0], b2hi[0]) - jnp.maximum(lo1[0], b2lo[0]), 0.0)
                iw1 = jnp.maximum(jnp.minimum(hi1[1], b2hi[1]) - jnp.maximum(lo1[1], b2lo[1]), 0.0)
                iw2 = jnp.maximum(jnp.minimum(hi1[2], b2hi[2]) - jnp.maximum(lo1[2], b2lo[2]), 0.0)
                inter = iw0 * iw1 * iw2
                union = v1 + v2 - inter
                iou = inter / (union + 1e-9)
                ew0 = jnp.maximum(jnp.maximum(hi1[0], b2hi[0]) - jnp.minimum(lo1[0], b2lo[0]), 0.0)
                ew1 = jnp.maximum(jnp.maximum(hi1[1], b2hi[1]) - jnp.minimum(lo1[1], b2lo[1]), 0.0)
                ew2 = jnp.maximum(jnp.maximum(hi1[2], b2hi[2]) - jnp.minimum(lo1[2], b2lo[2]), 0.0)
                enc = ew0 * ew1 * ew2
                giou = iou - (enc - union) / (enc + 1e-9)
                costg = -giou

                sig = 1.0 / (1.0 + jnp.exp(-p))
                c_all = bbox - sig + costg + center

                costg_v[pl.ds(qoff, _L)] = costg

                better = c_all < minval
                qv = jnp.full((_L,), q, dtype=jnp.int32)
                minval = jnp.where(better, c_all, minval)
                minidx = jnp.where(better, qv, minidx)
                gmin = jnp.minimum(gmin, costg)
                gmax = jnp.maximum(gmax, costg)
                return (minval, minidx, gmin, gmax)

            _mv, minidx, gmin, gmax = lax.fori_loop(
                0, _NQ, _q, (inf, zero_i, inf, ninf))
            del _mv

            denom = gmin - gmax - 1e-12
            sidx = minidx * ntq

            def _out(q, c1):
                iv = cbase + q
                g = costg_v[pl.ds(q * _CPAD + hoff, _L)]
                soft = jnp.maximum((g - gmax) / denom, 0.0)
                m = jnp.where(sidx == q, 1, 0).astype(jnp.int32)
                if mask is None:
                    plsc.store_scatter(soft_v, [iv], soft)
                    plsc.store_scatter(match_v, [iv], m)
                else:
                    plsc.store_scatter(soft_v, [iv], soft, mask=mask)
                    plsc.store_scatter(match_v, [iv], m, mask=mask)
                return c1

            lax.fori_loop(0, _NQ, _out, 0)
        return c0

    lax.fori_loop(0, _BPW, _batch, 0)

    pltpu.sync_copy(soft_v, soft_out.at[pl.ds(base * _FLAT, _BPW * _FLAT)])
    pltpu.sync_copy(match_v, match_out.at[pl.ds(base * _FLAT, _BPW * _FLAT)])


@jax.jit
def _matcher(probs, tgt_t, anch_t, ntq_arr):
    mesh = plsc.VectorSubcoreMesh(core_axis_name="c", subcore_axis_name="s")
    k = functools.partial(
        pl.kernel,
        mesh=mesh,
        compiler_params=pltpu.CompilerParams(needs_layout_passes=False),
        out_type=[
            jax.ShapeDtypeStruct((_BS * _FLAT,), jnp.float32),
            jax.ShapeDtypeStruct((_BS * _FLAT,), jnp.int32),
        ],
        scratch_types=[
            pltpu.VMEM((_BPW * _FLAT,), jnp.float32),
            pltpu.VMEM((_BPW * _TGT_W,), jnp.float32),
            pltpu.VMEM((6 * _ANCH_W,), jnp.float32),
            pltpu.VMEM((3 * _ANCH_W,), jnp.float32),
            pltpu.VMEM((3 * _ANCH_W,), jnp.float32),
            pltpu.VMEM((_ANCH_W,), jnp.float32),
            pltpu.VMEM((_ANCH_W,), jnp.float32),
            pltpu.VMEM((_BPW * _FLAT,), jnp.float32),
            pltpu.VMEM((_BPW * _FLAT,), jnp.int32),
            pltpu.VMEM((_L,), jnp.int32),
        ],
    )(_sc_body)
    return k(probs, tgt_t, anch_t, ntq_arr)


def kernel(pred_logits, pred_boxes, anchors, target_boxes, target_labels,
           num_top_queries):
    del pred_boxes, target_labels
    bs = pred_logits.shape[0]
    probs = pred_logits.reshape(bs * _FLAT)
    tgt_t = jnp.pad(jnp.transpose(target_boxes, (0, 2, 1)),
                    ((0, 0), (0, 0), (0, _CPAD - _NCLS))).reshape(-1)
    anch_t = jnp.pad(
        jnp.transpose(anchors.reshape(_NCLS, _NQ, 6), (2, 1, 0)),
        ((0, 0), (0, 0), (0, _CPAD - _NCLS))).reshape(-1)
    ntq_arr = jnp.full((_L,), num_top_queries, dtype=jnp.int32)
    soft, match_flat = _matcher(probs, tgt_t, anch_t, ntq_arr)
    return match_flat.reshape(bs, _NCLS, _NQ), soft.reshape(bs, _FLAT)
